# baseline (speedup 1.0000x reference)
import functools

import jax
import jax.numpy as jnp
from jax import lax
from jax.experimental import pallas as pl
from jax.experimental.pallas import tpu as pltpu
from jax.experimental.pallas import tpu_sc as plsc


@functools.partial(jax.jit, static_argnames=("ch", "nbuf"))
def _lookup(idx3, table, ch, nbuf):
    nw, n_ch, _ = idx3.shape
    _, D = table.shape
    info = plsc.get_sparse_core_info()
    nc = info.num_cores
    ns = nw // nc
    b_per_w = n_ch * ch
    half = (nw // nc) * b_per_w
    n_groups = n_ch // nbuf
    assert n_groups * nbuf == n_ch

    mesh = plsc.VectorSubcoreMesh(core_axis_name="c", subcore_axis_name="s")

    @functools.partial(
        pl.kernel,
        mesh=mesh,
        out_type=(jax.ShapeDtypeStruct((half, D), jnp.float32),
                  jax.ShapeDtypeStruct((half, D), jnp.float32)),
        compiler_params=pltpu.CompilerParams(use_tc_tiling_on_sc=False),
        scratch_types=(
            [pltpu.VMEM((n_ch, ch), jnp.int32),
             pltpu.VMEM((nbuf, ch, D), jnp.float32)]
            + [pltpu.SemaphoreType.DMA] * (2 * nbuf)
        ),
    )
    def k(idx_hbm, table_hbm, out0_hbm, out1_hbm, idx_v, rows_v, *sems):
        g_sems, s_sems = sems[:nbuf], sems[nbuf:]
        cid = lax.axis_index("c")
        sid = lax.axis_index("s")
        wid = cid * ns + sid
        base = sid * b_per_w

        def gather(b, j):
            return pltpu.make_async_copy(
                table_hbm.at[idx_v.at[j]], rows_v.at[b], g_sems[b])

        def store(out_hbm, b, c):
            return pltpu.make_async_copy(
                rows_v.at[b], out_hbm.at[pl.ds(base + c * ch, ch)], s_sems[b])

        pltpu.sync_copy(idx_hbm.at[wid], idx_v)
        for b in range(nbuf):
            gather(b, b).start()

        def run(out_hbm):
            def body(g, carry):
                c0 = g * nbuf
                for b in range(nbuf):
                    gather(b, c0 + b).wait()
                    store(out_hbm, b, c0 + b).start()
                for b in range(nbuf):
                    cn = c0 + b + nbuf

                    @pl.when(cn < n_ch)
                    def _():
                        store(out_hbm, b, 0).wait()
                        gather(b, cn).start()

                return carry

            lax.fori_loop(0, n_groups, body, 0)
            for b in range(nbuf):
                store(out_hbm, b, 0).wait()

        @pl.when(cid == 0)
        def _():
            run(out0_hbm)

        @pl.when(cid == 1)
        def _():
            run(out1_hbm)

    return k(idx3, table)


def kernel(input, table):
    b0, b1 = input.shape
    d = table.shape[1]
    ch, nbuf = 512, 2
    idx = input.reshape(-1).astype(jnp.int32)
    nw = 32
    n_ch = idx.shape[0] // (nw * ch)
    idx3 = idx.reshape(nw, n_ch, ch)
    out0, out1 = _lookup(idx3, table, ch, nbuf)
    out = jnp.concatenate([out0, out1], axis=0)
    return out.reshape(b0, b1, d)


# speedup vs baseline: 1.3625x; 1.3625x over previous
import functools

import jax
import jax.numpy as jnp
from jax import lax
from jax.experimental import pallas as pl
from jax.experimental.pallas import tpu as pltpu
from jax.experimental.pallas import tpu_sc as plsc


@functools.partial(jax.jit, static_argnames=("ch", "nbuf"))
def _lookup(idx3, table, ch, nbuf):
    nw, n_ch, _ = idx3.shape
    B = nw * n_ch * ch
    _, D = table.shape
    info = plsc.get_sparse_core_info()
    nc = info.num_cores
    b_per_w = n_ch * ch
    n_groups = n_ch // nbuf
    assert n_groups * nbuf == n_ch

    mesh = plsc.VectorSubcoreMesh(core_axis_name="c", subcore_axis_name="s")

    @functools.partial(
        pl.kernel,
        mesh=mesh,
        out_type=jax.ShapeDtypeStruct((B, D), jnp.float32),
        compiler_params=pltpu.CompilerParams(use_tc_tiling_on_sc=False),
        scratch_types=(
            [pltpu.VMEM((n_ch, ch), jnp.int32),
             pltpu.VMEM((nbuf, ch, D), jnp.float32)]
            + [pltpu.SemaphoreType.DMA] * (2 * nbuf)
        ),
    )
    def k(idx_hbm, table_hbm, out_hbm, idx_v, rows_v, *sems):
        g_sems, s_sems = sems[:nbuf], sems[nbuf:]
        wid = lax.axis_index("s") * nc + lax.axis_index("c")
        base = wid * b_per_w

        def gather(b, j):
            return pltpu.make_async_copy(
                table_hbm.at[idx_v.at[j]], rows_v.at[b], g_sems[b])

        def store(b, c):
            return pltpu.make_async_copy(
                rows_v.at[b], out_hbm.at[pl.ds(base + c * ch, ch)], s_sems[b])

        pltpu.sync_copy(idx_hbm.at[wid], idx_v)
        for b in range(nbuf):
            gather(b, b).start()

        def body(g, carry):
            c0 = g * nbuf
            for b in range(nbuf):
                gather(b, c0 + b).wait()
                store(b, c0 + b).start()
            for b in range(nbuf):
                cn = c0 + b + nbuf

                @pl.when(cn < n_ch)
                def _():
                    store(b, 0).wait()
                    gather(b, cn).start()

            return carry

        lax.fori_loop(0, n_groups, body, 0)
        for b in range(nbuf):
            store(b, 0).wait()

    return k(idx3, table)


def kernel(input, table):
    b0, b1 = input.shape
    d = table.shape[1]
    ch, nbuf = 256, 4
    idx = input.reshape(-1).astype(jnp.int32)
    nw = 32
    n_ch = idx.shape[0] // (nw * ch)
    out = _lookup(idx.reshape(nw, n_ch, ch), table, ch, nbuf)
    return out.reshape(b0, b1, d)


# speedup vs baseline: 1.3637x; 1.0009x over previous
import functools

import jax
import jax.numpy as jnp
from jax import lax
from jax.experimental import pallas as pl
from jax.experimental.pallas import tpu as pltpu
from jax.experimental.pallas import tpu_sc as plsc


@functools.partial(jax.jit, static_argnames=("ch", "nbuf"))
def _lookup(idx3, table, ch, nbuf):
    nw, n_ch, _ = idx3.shape
    B = nw * n_ch * ch
    _, D = table.shape
    info = plsc.get_sparse_core_info()
    nc = info.num_cores
    b_per_w = n_ch * ch
    n_groups = n_ch // nbuf
    assert n_groups * nbuf == n_ch

    mesh = plsc.VectorSubcoreMesh(core_axis_name="c", subcore_axis_name="s")

    @functools.partial(
        pl.kernel,
        mesh=mesh,
        out_type=jax.ShapeDtypeStruct((B, D), jnp.float32),
        compiler_params=pltpu.CompilerParams(use_tc_tiling_on_sc=False),
        scratch_types=(
            [pltpu.VMEM((n_ch, ch), jnp.int32),
             pltpu.VMEM((nbuf, ch, D), jnp.float32)]
            + [pltpu.SemaphoreType.DMA] * (2 * nbuf)
        ),
    )
    def k(idx_hbm, table_hbm, out_hbm, idx_v, rows_v, *sems):
        g_sems, s_sems = sems[:nbuf], sems[nbuf:]
        wid = lax.axis_index("s") * nc + lax.axis_index("c")
        base = wid * b_per_w

        def gather(b, j):
            return pltpu.make_async_copy(
                table_hbm.at[idx_v.at[j]], rows_v.at[b], g_sems[b])

        def store(b, c):
            return pltpu.make_async_copy(
                rows_v.at[b], out_hbm.at[pl.ds(base + c * ch, ch)], s_sems[b])

        pltpu.sync_copy(idx_hbm.at[wid], idx_v)
        for b in range(nbuf):
            gather(b, b).start()

        def body(g, carry):
            c0 = g * nbuf
            for b in range(nbuf):
                gather(b, c0 + b).wait()
                store(b, c0 + b).start()
            for b in range(nbuf):
                cn = c0 + b + nbuf

                @pl.when(cn < n_ch)
                def _():
                    store(b, 0).wait()
                    gather(b, cn).start()

            return carry

        lax.fori_loop(0, n_groups, body, 0)
        for b in range(nbuf):
            store(b, 0).wait()

    return k(idx3, table)


def kernel(input, table):
    b0, b1 = input.shape
    d = table.shape[1]
    ch, nbuf = 128, 8
    idx = input.reshape(-1).astype(jnp.int32)
    nw = 32
    n_ch = idx.shape[0] // (nw * ch)
    out = _lookup(idx.reshape(nw, n_ch, ch), table, ch, nbuf)
    return out.reshape(b0, b1, d)
